# baseline (speedup 1.0000x reference)
import jax
import jax.numpy as jnp
from jax.experimental import pallas as pl

K = 8
W_CONST = 0.01
EPS = 1e-8
N_V = 16384
BN = 128


def _body(xa_ref, x_ref, vt_ref, Rh_ref, Rl_ref, o_ref):
    xa = xa_ref[...]
    vt = vt_ref[...]
    mm = jnp.dot(xa, vt, preferred_element_type=jnp.float32)
    e = vt[4:5, :] + mm

    vx = vt[0:1, :]
    vy = vt[1:2, :]
    vz = vt[2:3, :]
    h = N_V // 2
    c = e[:, :h] <= e[:, h:]
    m = jnp.where(c, e[:, :h], e[:, h:])
    s = jnp.where(c, e[:, h:], e[:, :h])
    px = jnp.where(c, vx[:, :h], vx[:, h:])
    py = jnp.where(c, vy[:, :h], vy[:, h:])
    pz = jnp.where(c, vz[:, :h], vz[:, h:])
    width = h
    for _ in range(2):
        q = width // 2
        mL, mR = m[:, :q], m[:, q:]
        c = mL <= mR
        s = jnp.minimum(jnp.where(c, mR, mL),
                        jnp.minimum(s[:, :q], s[:, q:]))
        m = jnp.where(c, mL, mR)
        px = jnp.where(c, px[:, :q], px[:, q:])
        py = jnp.where(c, py[:, :q], py[:, q:])
        pz = jnp.where(c, pz[:, :q], pz[:, q:])
        width = q
    cand = jnp.concatenate([m, s], axis=1)

    while width > 1:
        q = width // 2
        mL, mR = m[:, :q], m[:, q:]
        c = mL <= mR
        m = jnp.where(c, mL, mR)
        px = jnp.where(c, px[:, :q], px[:, q:])
        py = jnp.where(c, py[:, :q], py[:, q:])
        pz = jnp.where(c, pz[:, :q], pz[:, q:])
        width = q
    mn = m
    v1 = jnp.concatenate([px, py, pz], axis=1)

    m_prev = mn
    for k in range(K - 1):
        cur = jnp.where(cand > m_prev, cand, jnp.inf)
        m_prev = jnp.min(cur, axis=1, keepdims=True)
    t8 = m_prev

    d2 = e + xa[:, 3:4]
    inv = 1.0 / jnp.maximum(d2, EPS)
    w = jnp.where(e <= t8, inv, 0.0)

    Rh = Rh_ref[...]
    Rl = Rl_ref[...]
    wh = w.astype(jnp.bfloat16)
    wl = (w - wh.astype(jnp.float32)).astype(jnp.bfloat16)
    P1 = (jnp.dot(wh, Rh, preferred_element_type=jnp.float32)
          + jnp.dot(wh, Rl, preferred_element_type=jnp.float32)
          + jnp.dot(wl, Rh, preferred_element_type=jnp.float32))

    term_knn = P1[:, 0:3]
    Wk = P1[:, 3:4]

    x = x_ref[...]
    dv = x - v1
    d2v1 = jnp.maximum(jnp.sum(dv * dv, axis=1, keepdims=True), EPS)
    term_dir = dv / (W_CONST * d2v1)
    W = Wk + 1.0 / W_CONST
    n_tilde = (term_knn + term_dir) / W
    nrm = jnp.sqrt(jnp.sum(n_tilde * n_tilde, axis=1, keepdims=True))
    nc = n_tilde / (nrm + 1e-8)
    s_ = jnp.sum(dv * nc, axis=1, keepdims=True)
    xc = x - s_ * nc
    o_ref[...] = jnp.concatenate([xc, s_, nc], axis=1)


def kernel(x, vertices, vertex_normals):
    x = x.astype(jnp.float32)
    vertices = vertices.astype(jnp.float32)
    vertex_normals = vertex_normals.astype(jnp.float32)

    n = x.shape[0]
    v = vertices.shape[0]

    x2 = jnp.sum(x * x, axis=1, keepdims=True)
    xa = jnp.concatenate([-2.0 * x, x2, jnp.zeros((n, 4), jnp.float32)], axis=1)
    v2 = jnp.sum(vertices * vertices, axis=1)
    vt = jnp.concatenate(
        [vertices.T, jnp.zeros((1, v), jnp.float32), v2[None, :], jnp.zeros((3, v), jnp.float32)],
        axis=0,
    )
    R = jnp.concatenate(
        [vertex_normals, jnp.ones((v, 1), jnp.float32), jnp.zeros((v, 124), jnp.float32)],
        axis=1,
    )
    Rh = R.astype(jnp.bfloat16)
    Rl = (R - Rh.astype(jnp.float32)).astype(jnp.bfloat16)

    grid = (n // BN,)
    out = pl.pallas_call(
        _body,
        grid=grid,
        in_specs=[
            pl.BlockSpec((BN, 8), lambda i: (i, 0)),
            pl.BlockSpec((BN, 3), lambda i: (i, 0)),
            pl.BlockSpec((8, N_V), lambda i: (0, 0)),
            pl.BlockSpec((N_V, 128), lambda i: (0, 0)),
            pl.BlockSpec((N_V, 128), lambda i: (0, 0)),
        ],
        out_specs=pl.BlockSpec((BN, 7), lambda i: (i, 0)),
        out_shape=jax.ShapeDtypeStruct((n, 7), jnp.float32),
    )(xa, x, vt, Rh, Rl)
    return out


# speedup vs baseline: 1.0312x; 1.0312x over previous
import jax
import jax.numpy as jnp
from jax.experimental import pallas as pl

K = 8
W_CONST = 0.01
EPS = 1e-8
N_V = 16384
BN = 256


def _body(xa_ref, x_ref, vt_ref, Rh_ref, Rl_ref, o_ref):
    xa = xa_ref[...]
    vt = vt_ref[...]
    mm = jnp.dot(xa, vt, preferred_element_type=jnp.float32)
    e = vt[4:5, :] + mm

    vx = vt[0:1, :]
    vy = vt[1:2, :]
    vz = vt[2:3, :]
    h = N_V // 2
    c = e[:, :h] <= e[:, h:]
    m = jnp.where(c, e[:, :h], e[:, h:])
    s = jnp.where(c, e[:, h:], e[:, :h])
    px = jnp.where(c, vx[:, :h], vx[:, h:])
    py = jnp.where(c, vy[:, :h], vy[:, h:])
    pz = jnp.where(c, vz[:, :h], vz[:, h:])
    width = h
    for _ in range(2):
        q = width // 2
        mL, mR = m[:, :q], m[:, q:]
        c = mL <= mR
        s = jnp.minimum(jnp.where(c, mR, mL),
                        jnp.minimum(s[:, :q], s[:, q:]))
        m = jnp.where(c, mL, mR)
        px = jnp.where(c, px[:, :q], px[:, q:])
        py = jnp.where(c, py[:, :q], py[:, q:])
        pz = jnp.where(c, pz[:, :q], pz[:, q:])
        width = q
    cand = jnp.concatenate([m, s], axis=1)

    while width > 1:
        q = width // 2
        mL, mR = m[:, :q], m[:, q:]
        c = mL <= mR
        m = jnp.where(c, mL, mR)
        px = jnp.where(c, px[:, :q], px[:, q:])
        py = jnp.where(c, py[:, :q], py[:, q:])
        pz = jnp.where(c, pz[:, :q], pz[:, q:])
        width = q
    mn = m
    v1 = jnp.concatenate([px, py, pz], axis=1)

    m_prev = mn
    for k in range(K - 1):
        cur = jnp.where(cand > m_prev, cand, jnp.inf)
        m_prev = jnp.min(cur, axis=1, keepdims=True)
    t8 = m_prev

    d2 = e + xa[:, 3:4]
    inv = 1.0 / jnp.maximum(d2, EPS)
    w = jnp.where(e <= t8, inv, 0.0)

    Rh = Rh_ref[...]
    Rl = Rl_ref[...]
    wh = w.astype(jnp.bfloat16)
    wl = (w - wh.astype(jnp.float32)).astype(jnp.bfloat16)
    P1 = (jnp.dot(wh, Rh, preferred_element_type=jnp.float32)
          + jnp.dot(wh, Rl, preferred_element_type=jnp.float32)
          + jnp.dot(wl, Rh, preferred_element_type=jnp.float32))

    term_knn = P1[:, 0:3]
    Wk = P1[:, 3:4]

    x = x_ref[...]
    dv = x - v1
    d2v1 = jnp.maximum(jnp.sum(dv * dv, axis=1, keepdims=True), EPS)
    term_dir = dv / (W_CONST * d2v1)
    W = Wk + 1.0 / W_CONST
    n_tilde = (term_knn + term_dir) / W
    nrm = jnp.sqrt(jnp.sum(n_tilde * n_tilde, axis=1, keepdims=True))
    nc = n_tilde / (nrm + 1e-8)
    s_ = jnp.sum(dv * nc, axis=1, keepdims=True)
    xc = x - s_ * nc
    o_ref[...] = jnp.concatenate([xc, s_, nc], axis=1)


def kernel(x, vertices, vertex_normals):
    x = x.astype(jnp.float32)
    vertices = vertices.astype(jnp.float32)
    vertex_normals = vertex_normals.astype(jnp.float32)

    n = x.shape[0]
    v = vertices.shape[0]

    x2 = jnp.sum(x * x, axis=1, keepdims=True)
    xa = jnp.concatenate([-2.0 * x, x2, jnp.zeros((n, 4), jnp.float32)], axis=1)
    v2 = jnp.sum(vertices * vertices, axis=1)
    vt = jnp.concatenate(
        [vertices.T, jnp.zeros((1, v), jnp.float32), v2[None, :], jnp.zeros((3, v), jnp.float32)],
        axis=0,
    )
    R = jnp.concatenate(
        [vertex_normals, jnp.ones((v, 1), jnp.float32), jnp.zeros((v, 124), jnp.float32)],
        axis=1,
    )
    Rh = R.astype(jnp.bfloat16)
    Rl = (R - Rh.astype(jnp.float32)).astype(jnp.bfloat16)

    grid = (n // BN,)
    out = pl.pallas_call(
        _body,
        grid=grid,
        in_specs=[
            pl.BlockSpec((BN, 8), lambda i: (i, 0)),
            pl.BlockSpec((BN, 3), lambda i: (i, 0)),
            pl.BlockSpec((8, N_V), lambda i: (0, 0)),
            pl.BlockSpec((N_V, 128), lambda i: (0, 0)),
            pl.BlockSpec((N_V, 128), lambda i: (0, 0)),
        ],
        out_specs=pl.BlockSpec((BN, 7), lambda i: (i, 0)),
        out_shape=jax.ShapeDtypeStruct((n, 7), jnp.float32),
    )(xa, x, vt, Rh, Rl)
    return out


# speedup vs baseline: 1.3226x; 1.2826x over previous
import jax
import jax.numpy as jnp
from jax.experimental import pallas as pl

K = 8
W_CONST = 0.01
EPS = 1e-8
N_V = 16384
BN = 256


def _body(xa_ref, x_ref, vt_ref, Rcat_ref, o_ref):
    xa = xa_ref[...]
    vt = vt_ref[...]
    mm = jnp.dot(xa, vt, preferred_element_type=jnp.float32)
    e = vt[4:5, :] + mm

    vx = vt[0:1, :]
    vy = vt[1:2, :]
    vz = vt[2:3, :]
    h = N_V // 2
    c = e[:, :h] <= e[:, h:]
    m = jnp.where(c, e[:, :h], e[:, h:])
    s = jnp.where(c, e[:, h:], e[:, :h])
    px = jnp.where(c, vx[:, :h], vx[:, h:])
    py = jnp.where(c, vy[:, :h], vy[:, h:])
    pz = jnp.where(c, vz[:, :h], vz[:, h:])
    width = h
    for _ in range(3):
        q = width // 2
        mL, mR = m[:, :q], m[:, q:]
        c = mL <= mR
        s = jnp.minimum(jnp.where(c, mR, mL),
                        jnp.minimum(s[:, :q], s[:, q:]))
        m = jnp.where(c, mL, mR)
        px = jnp.where(c, px[:, :q], px[:, q:])
        py = jnp.where(c, py[:, :q], py[:, q:])
        pz = jnp.where(c, pz[:, :q], pz[:, q:])
        width = q
    cand = jnp.concatenate([m, s], axis=1)

    while width > 1:
        q = width // 2
        mL, mR = m[:, :q], m[:, q:]
        c = mL <= mR
        m = jnp.where(c, mL, mR)
        px = jnp.where(c, px[:, :q], px[:, q:])
        py = jnp.where(c, py[:, :q], py[:, q:])
        pz = jnp.where(c, pz[:, :q], pz[:, q:])
        width = q
    mn = m
    v1 = jnp.concatenate([px, py, pz], axis=1)

    m_prev = mn
    for k in range(K - 1):
        cur = jnp.where(cand > m_prev, cand, jnp.inf)
        m_prev = jnp.min(cur, axis=1, keepdims=True)
    t8 = m_prev

    d2 = e + xa[:, 3:4]
    w = jnp.where(e <= t8, 1.0, 0.0) / jnp.maximum(d2, EPS)

    Rcat = Rcat_ref[...]
    wh = w.astype(jnp.bfloat16)
    wl = (w - wh.astype(jnp.float32)).astype(jnp.bfloat16)
    Ph = jnp.dot(wh, Rcat, preferred_element_type=jnp.float32)
    Pl = jnp.dot(wl, Rcat[:, 0:128], preferred_element_type=jnp.float32)

    term_knn = Ph[:, 0:3] + Ph[:, 128:131] + Pl[:, 0:3]
    Wk = Ph[:, 3:4] + Ph[:, 131:132] + Pl[:, 3:4]

    x = x_ref[...]
    dv = x - v1
    d2v1 = jnp.maximum(jnp.sum(dv * dv, axis=1, keepdims=True), EPS)
    term_dir = dv / (W_CONST * d2v1)
    W = Wk + 1.0 / W_CONST
    n_tilde = (term_knn + term_dir) / W
    nrm = jnp.sqrt(jnp.sum(n_tilde * n_tilde, axis=1, keepdims=True))
    nc = n_tilde / (nrm + 1e-8)
    s_ = jnp.sum(dv * nc, axis=1, keepdims=True)
    xc = x - s_ * nc
    o_ref[...] = jnp.concatenate([xc, s_, nc], axis=1)


def kernel(x, vertices, vertex_normals):
    x = x.astype(jnp.float32)
    vertices = vertices.astype(jnp.float32)
    vertex_normals = vertex_normals.astype(jnp.float32)

    n = x.shape[0]
    v = vertices.shape[0]

    x2 = jnp.sum(x * x, axis=1, keepdims=True)
    xa = jnp.concatenate([-2.0 * x, x2, jnp.zeros((n, 4), jnp.float32)], axis=1)
    v2 = jnp.sum(vertices * vertices, axis=1)
    vt = jnp.concatenate(
        [vertices.T, jnp.zeros((1, v), jnp.float32), v2[None, :], jnp.zeros((3, v), jnp.float32)],
        axis=0,
    )
    R = jnp.concatenate(
        [vertex_normals, jnp.ones((v, 1), jnp.float32), jnp.zeros((v, 124), jnp.float32)],
        axis=1,
    )
    Rh = R.astype(jnp.bfloat16)
    Rl = (R - Rh.astype(jnp.float32)).astype(jnp.bfloat16)
    Rcat = jnp.concatenate([Rh, Rl], axis=1)

    grid = (n // BN,)
    out = pl.pallas_call(
        _body,
        grid=grid,
        in_specs=[
            pl.BlockSpec((BN, 8), lambda i: (i, 0)),
            pl.BlockSpec((BN, 3), lambda i: (i, 0)),
            pl.BlockSpec((8, N_V), lambda i: (0, 0)),
            pl.BlockSpec((N_V, 256), lambda i: (0, 0)),
        ],
        out_specs=pl.BlockSpec((BN, 7), lambda i: (i, 0)),
        out_shape=jax.ShapeDtypeStruct((n, 7), jnp.float32),
    )(xa, x, vt, Rcat)
    return out
